# baseline (speedup 1.0000x reference)
import functools

import jax
import jax.numpy as jnp
from jax import lax
from jax.experimental import pallas as pl
from jax.experimental.pallas import tpu as pltpu
from jax.experimental.pallas import tpu_sc as plsc

_info = plsc.get_sparse_core_info()
_NC = _info.num_cores
_NS = _info.num_subcores
_NW = _NC * _NS
_L = _info.num_lanes

_ROWS = 10
_D = 20
_W = 10
_PT = _ROWS * _ROWS * _D
_BT = 128
_DC = 4


def _sc_lookup(b: int, s: int):
    n_ps = s // 2
    n_blk = b // (_NW * _BT)
    n_dc = _D // _DC
    assert b % (_NW * _BT) == 0 and s % 2 == 0

    mesh = plsc.VectorSubcoreMesh(core_axis_name="c", subcore_axis_name="s")

    @functools.partial(
        pl.kernel,
        mesh=mesh,
        out_type=jax.ShapeDtypeStruct((2 * _D * n_ps, b), jnp.float16),
        compiler_params=pltpu.CompilerParams(
            needs_layout_passes=False,
            disable_bounds_checks=True,
        ),
        scratch_types=[
            pltpu.VMEM((_ROWS * _W,), jnp.int32),
            pltpu.VMEM((_PT,), jnp.int32),
            pltpu.VMEM((s, _BT), jnp.int32),
            pltpu.VMEM((_DC * n_ps, _BT), jnp.int32),
            pltpu.VMEM((_DC * n_ps, _BT), jnp.int32),
            pltpu.SemaphoreType.DMA,
            pltpu.SemaphoreType.DMA,
        ],
    )
    def body(table_hbm, idxt_hbm, out_hbm, table_v, ptab_v,
             idx_v, out_v0, out_v1, out_sem0, out_sem1):
        wid = lax.axis_index("s") * _NC + lax.axis_index("c")
        out_words = out_hbm.bitcast(jnp.int32)
        pltpu.sync_copy(table_hbm, table_v)

        lane = lax.iota(jnp.int32, _L)

        def ptab_body(j):
            pos = j * _L + lane
            pidx = pos // _D
            d = pos - pidx * _D
            i0 = pidx // _ROWS
            i1 = pidx - i0 * _ROWS
            w = d // 2
            sh = (d - w * 2) * 16
            w0 = plsc.load_gather(table_v, [i0 * _W + w])
            w1 = plsc.load_gather(table_v, [i1 * _W + w])
            v0 = lax.shift_right_logical(w0, sh) & 0xFFFF
            v1 = lax.shift_right_logical(w1, sh) & 0xFFFF
            plsc.store_scatter(ptab_v, [pos], v0 | lax.shift_left(v1, 16))

        plsc.parallel_loop(0, _PT // _L, unroll=4)(ptab_body)

        out_bufs = (out_v0, out_v1)
        out_sems = (out_sem0, out_sem1)

        def idx_src(k):
            return idxt_hbm.at[:, pl.ds(wid * (n_blk * _BT) + k * _BT, _BT)]

        def out_dst(k, dc):
            return out_words.at[
                pl.ds(dc * (_DC * n_ps), _DC * n_ps),
                pl.ds(wid * (n_blk * _BT) + k * _BT, _BT),
            ]

        n_q = n_blk * n_dc

        def q_pair(q2, carry):
            for qq in range(2):
                q = q2 * 2 + qq
                k = q // n_dc
                dc = q - k * n_dc
                out_v = out_bufs[qq]

                @pl.when(dc == 0)
                def _():
                    pltpu.sync_copy(idx_src(k), idx_v)

                    def pidx_body(ps):
                        for l in range(_BT // _L):
                            e16 = idx_v[2 * ps, pl.ds(l * _L, _L)]
                            o16 = idx_v[2 * ps + 1, pl.ds(l * _L, _L)]
                            idx_v[2 * ps, pl.ds(l * _L, _L)] = (
                                (e16 * _ROWS + o16) * _D
                            )

                    plsc.parallel_loop(0, n_ps, unroll=4)(pidx_body)

                @pl.when(q >= 2)
                def _():
                    pltpu.make_async_copy(
                        out_v, out_dst(0, 0), out_sems[qq]
                    ).wait()

                pd = dc * _DC

                def gat_body(ps):
                    for l in range(_BT // _L):
                        p20 = idx_v[2 * ps, pl.ds(l * _L, _L)] + pd
                        for dd in range(_DC):
                            val = plsc.load_gather(ptab_v, [p20 + dd])
                            out_v[dd * n_ps + ps, pl.ds(l * _L, _L)] = val

                plsc.parallel_loop(0, n_ps, unroll=4)(gat_body)
                pltpu.async_copy(out_v, out_dst(k, dc), out_sems[qq])
            return carry

        lax.fori_loop(0, n_q // 2, q_pair, 0)
        for q in (n_q - 2, n_q - 1):
            pltpu.make_async_copy(
                out_bufs[q % 2], out_dst(0, 0), out_sems[q % 2]
            ).wait()

    return body


def kernel(x, table):
    b, s = x.shape
    rows, d = table.shape
    idx_t = x.T.astype(jnp.int32)
    table_words = lax.bitcast_convert_type(
        table.reshape(rows * d // 2, 2), jnp.int32
    )
    y = _sc_lookup(b, s)(table_words, idx_t)
    return y.reshape(d, s, b).transpose(2, 1, 0)


# speedup vs baseline: 1.0313x; 1.0313x over previous
import functools

import jax
import jax.numpy as jnp
from jax import lax
from jax.experimental import pallas as pl
from jax.experimental.pallas import tpu as pltpu
from jax.experimental.pallas import tpu_sc as plsc

_info = plsc.get_sparse_core_info()
_NC = _info.num_cores
_NS = _info.num_subcores
_NW = _NC * _NS
_L = _info.num_lanes

_ROWS = 10
_D = 20
_W = 10
_PT = _ROWS * _ROWS * _D
_BT = 128
_DC = 4


def _sc_lookup(b: int, s: int):
    n_ps = s // 2
    n_blk = b // (_NW * _BT)
    n_dc = _D // _DC
    assert b % (_NW * _BT) == 0 and s % 2 == 0

    mesh = plsc.VectorSubcoreMesh(core_axis_name="c", subcore_axis_name="s")

    @functools.partial(
        pl.kernel,
        mesh=mesh,
        out_type=jax.ShapeDtypeStruct((2 * _D * n_ps, b), jnp.float16),
        compiler_params=pltpu.CompilerParams(
            needs_layout_passes=False,
            disable_bounds_checks=True,
        ),
        scratch_types=[
            pltpu.VMEM((_ROWS * _W,), jnp.int32),
            pltpu.VMEM((_PT,), jnp.int32),
            pltpu.VMEM((s, _BT), jnp.int32),
            pltpu.VMEM((_DC * n_ps, _BT), jnp.int32),
            pltpu.VMEM((_DC * n_ps, _BT), jnp.int32),
            pltpu.SemaphoreType.DMA,
            pltpu.SemaphoreType.DMA,
        ],
    )
    def body(table_hbm, idxt_hbm, out_hbm, table_v, ptab_v,
             idx_v, out_v0, out_v1, out_sem0, out_sem1):
        wid = lax.axis_index("s") * _NC + lax.axis_index("c")
        out_words = out_hbm.bitcast(jnp.int32)
        pltpu.sync_copy(table_hbm, table_v)

        lane = lax.iota(jnp.int32, _L)

        def ptab_body(j):
            pos = j * _L + lane
            pidx = pos // _D
            d = pos - pidx * _D
            i0 = pidx // _ROWS
            i1 = pidx - i0 * _ROWS
            w = d // 2
            sh = (d - w * 2) * 16
            w0 = plsc.load_gather(table_v, [i0 * _W + w])
            w1 = plsc.load_gather(table_v, [i1 * _W + w])
            v0 = lax.shift_right_logical(w0, sh) & 0xFFFF
            v1 = lax.shift_right_logical(w1, sh) & 0xFFFF
            plsc.store_scatter(ptab_v, [pos], v0 | lax.shift_left(v1, 16))

        plsc.parallel_loop(0, _PT // _L, unroll=4)(ptab_body)

        out_bufs = (out_v0, out_v1)
        out_sems = (out_sem0, out_sem1)

        def idx_src(k):
            return idxt_hbm.at[:, pl.ds(wid * (n_blk * _BT) + k * _BT, _BT)]

        def out_dst(k, dc):
            return out_words.at[
                pl.ds(dc * (_DC * n_ps), _DC * n_ps),
                pl.ds(wid * (n_blk * _BT) + k * _BT, _BT),
            ]

        n_q = n_blk * n_dc

        def q_pair(q2, carry):
            for qq in range(2):
                q = q2 * 2 + qq
                k = q // n_dc
                dc = q - k * n_dc
                out_v = out_bufs[qq]

                @pl.when(dc == 0)
                def _():
                    pltpu.sync_copy(idx_src(k), idx_v)

                    def pidx_body(ps):
                        for l in range(_BT // _L):
                            e16 = idx_v[2 * ps, pl.ds(l * _L, _L)]
                            o16 = idx_v[2 * ps + 1, pl.ds(l * _L, _L)]
                            idx_v[2 * ps, pl.ds(l * _L, _L)] = (
                                (e16 * _ROWS + o16) * _D
                            )

                    plsc.parallel_loop(0, n_ps, unroll=4)(pidx_body)

                @pl.when(q >= 2)
                def _():
                    pltpu.make_async_copy(
                        out_v, out_dst(0, 0), out_sems[qq]
                    ).wait()

                pd = dc * _DC

                def gat_body(ps):
                    for l in range(_BT // _L):
                        p20 = idx_v[2 * ps, pl.ds(l * _L, _L)] + pd
                        for dd in range(_DC):
                            val = plsc.load_gather(ptab_v, [p20 + dd])
                            out_v[dd * n_ps + ps, pl.ds(l * _L, _L)] = val

                plsc.parallel_loop(0, n_ps, unroll=2)(gat_body)
                pltpu.async_copy(out_v, out_dst(k, dc), out_sems[qq])
            return carry

        lax.fori_loop(0, n_q // 2, q_pair, 0)
        for q in (n_q - 2, n_q - 1):
            pltpu.make_async_copy(
                out_bufs[q % 2], out_dst(0, 0), out_sems[q % 2]
            ).wait()

    return body


def kernel(x, table):
    b, s = x.shape
    rows, d = table.shape
    idx_t = x.T.astype(jnp.int32)
    table_words = lax.bitcast_convert_type(
        table.reshape(rows * d // 2, 2), jnp.int32
    )
    y = _sc_lookup(b, s)(table_words, idx_t)
    return y.reshape(d, s, b).transpose(2, 1, 0)
